# baseline (speedup 1.0000x reference)
import functools

import jax
import jax.numpy as jnp
from jax import lax
from jax.experimental import pallas as pl
from jax.experimental.pallas import tpu as pltpu
from jax.experimental.pallas import tpu_sc as plsc

B = 16384
D = 64
NC = 2
NS = 16
NW = NC * NS
BPW = B // NW
K = 16
NCHUNK = BPW // K
VEC = 16

_mesh = plsc.VectorSubcoreMesh(core_axis_name="c", subcore_axis_name="s")


@functools.partial(
    pl.kernel,
    mesh=_mesh,
    out_type=jax.ShapeDtypeStruct((B, 2 * D), jnp.float32),
    scratch_types=[
        pltpu.VMEM((BPW,), jnp.int32),
        pltpu.VMEM((BPW,), jnp.int32),
        pltpu.SMEM((BPW,), jnp.int32),
        pltpu.SMEM((BPW,), jnp.int32),
        pltpu.VMEM((K * 8, D), jnp.float32),
        pltpu.VMEM((K * 8, D), jnp.float32),
        pltpu.VMEM((K * 8, D), jnp.float32),
        pltpu.VMEM((K * 8, D), jnp.float32),
        pltpu.VMEM((K, 2 * D), jnp.float32),
        pltpu.VMEM((K, 2 * D), jnp.float32),
        pltpu.SemaphoreType.DMA,
        pltpu.SemaphoreType.DMA,
        pltpu.SemaphoreType.DMA,
        pltpu.SemaphoreType.DMA,
    ],
)
def _emb_kernel(uid_hbm, iid_hbm, tuid_hbm, tiid_hbm, out_hbm,
                uidx_v, iidx_v, us_s, is_s,
                ugrp_a, ugrp_b, igrp_a, igrp_b, crows_a, crows_b,
                sem_a, sem_b, sem_wa, sem_wb):
    wid = lax.axis_index("s") * NC + lax.axis_index("c")
    base = wid * BPW

    pltpu.sync_copy(uid_hbm.at[pl.ds(base, BPW)], uidx_v)
    pltpu.sync_copy(iid_hbm.at[pl.ds(base, BPW)], iidx_v)

    def _stage(t, carry):
        xu = uidx_v[pl.ds(t * VEC, VEC)]
        xi = iidx_v[pl.ds(t * VEC, VEC)]
        for j in range(VEC):
            us_s[t * VEC + j] = xu[j]
            is_s[t * VEC + j] = xi[j]
        return carry

    lax.fori_loop(0, BPW // VEC, _stage, 0)

    def _fire(c, ugrp, igrp, sem):
        cbase = c * K

        def body(j, carry):
            iu = us_s[cbase + j]
            ii = is_s[cbase + j]
            gu = pl.multiple_of((iu >> 3) << 3, 8)
            gi = pl.multiple_of((ii >> 3) << 3, 8)
            pltpu.async_copy(tuid_hbm.at[pl.ds(gu, 8)],
                             ugrp.at[pl.ds(j * 8, 8)], sem)
            pltpu.async_copy(tiid_hbm.at[pl.ds(gi, 8)],
                             igrp.at[pl.ds(j * 8, 8)], sem)
            return carry

        lax.fori_loop(0, K, body, 0)

    def _drain(ugrp, sem):
        def body(j, carry):
            pltpu.make_async_copy(tuid_hbm.at[pl.ds(0, 8)],
                                  ugrp.at[pl.ds(0, 8)], sem).wait()
            return carry

        lax.fori_loop(0, 2 * K, body, 0)

    def _extract(c, ugrp, igrp, crows):
        cbase = c * K

        def body(j, carry):
            su = j * 8 + (us_s[cbase + j] & 7)
            si = j * 8 + (is_s[cbase + j] & 7)
            for kk in range(D // VEC):
                crows[j, pl.ds(kk * VEC, VEC)] = (
                    ugrp[su, pl.ds(kk * VEC, VEC)])
                crows[j, pl.ds(D + kk * VEC, VEC)] = (
                    igrp[si, pl.ds(kk * VEC, VEC)])
            return carry

        lax.fori_loop(0, K, body, 0)

    def _wdesc(crows, sem_w, c):
        return pltpu.make_async_copy(
            crows, out_hbm.at[pl.ds(base + c * K, K)], sem_w)

    _fire(0, ugrp_a, igrp_a, sem_a)
    for c in range(NCHUNK):
        even = (c % 2 == 0)
        ugrp = ugrp_a if even else ugrp_b
        igrp = igrp_a if even else igrp_b
        crows = crows_a if even else crows_b
        sem = sem_a if even else sem_b
        sem_nxt = sem_b if even else sem_a
        sem_w = sem_wa if even else sem_wb
        if c + 1 < NCHUNK:
            _fire(c + 1, ugrp_b if even else ugrp_a,
                  igrp_b if even else igrp_a, sem_nxt)
        _drain(ugrp, sem)
        if c >= 2:
            _wdesc(crows, sem_w, c - 2).wait()
        _extract(c, ugrp, igrp, crows)
        _wdesc(crows, sem_w, c).start()
    _wdesc(crows_a, sem_wa, NCHUNK - 2).wait()
    _wdesc(crows_b, sem_wb, NCHUNK - 1).wait()


def kernel(uid, iid, table_uid, table_iid):
    uid = uid.astype(jnp.int32)
    iid = iid.astype(jnp.int32)
    return _emb_kernel(uid, iid, table_uid, table_iid)
